# Initial kernel scaffold; baseline (speedup 1.0000x reference)
import functools
import jax
import jax.numpy as jnp
from jax import lax
from jax.experimental import pallas as pl
from jax.experimental.pallas import tpu as pltpu
from jax.experimental.pallas import tpu_sc as plsc

N = 10000
E = 320000
H = 128
N_PAD = 10240
NW = 32
C = 128
NCHUNK = 79
EPW = C * NCHUNK
E_PAD = NW * EPW
ROWS_PER_TILE = N_PAD // 16


def _pre_body(nf_ref, w1a_ref, w1b_ref, b1_ref, ap_ref, bt_ref):
    nf = nf_ref[...]
    ap_ref[...] = jnp.dot(nf, w1a_ref[...], preferred_element_type=jnp.float32) + b1_ref[...]
    bt_ref[...] = jnp.dot(nf, w1b_ref[...], preferred_element_type=jnp.float32)


def _pre(nf_pad, w1a, w1b, b1):
    R = 512
    return pl.pallas_call(
        _pre_body,
        grid=(N_PAD // R,),
        in_specs=[
            pl.BlockSpec((R, H), lambda i: (i, 0)),
            pl.BlockSpec((H, H), lambda i: (0, 0)),
            pl.BlockSpec((H, H), lambda i: (0, 0)),
            pl.BlockSpec((1, H), lambda i: (0, 0)),
        ],
        out_specs=[
            pl.BlockSpec((R, H), lambda i: (i, 0)),
            pl.BlockSpec((R, H), lambda i: (i, 0)),
        ],
        out_shape=[
            jax.ShapeDtypeStruct((N_PAD, H), jnp.float32),
            jax.ShapeDtypeStruct((N_PAD, H), jnp.float32),
        ],
    )(nf_pad, w1a, w1b, b1)


def _sc_edge_body(ap_hbm, bt_hbm, pos_hbm, row_hbm, col_hbm, w1c_hbm,
                  s_out, c_out,
                  row_v, col_v, bufA, bufB, posr, posc, sbuf, ones_b, w1c_v,
                  dist_b, S_sp, C_sp, semA, semB, semC, semD):
    core = lax.axis_index("c")
    sub = lax.axis_index("s")
    wid = sub * 2 + core
    tbase = sub * ROWS_PER_TILE

    zero16 = jnp.zeros((16,), jnp.float32)

    def _zs(r, carry):
        for j in range(8):
            sbuf[r, pl.ds(16 * j, 16)] = zero16
        return carry
    lax.fori_loop(0, C, _zs, 0)
    for i in range(ROWS_PER_TILE // C):
        pltpu.sync_copy(sbuf, S_sp.at[pl.ds(tbase + C * i, C)])

    def _zp(r, carry):
        posr[r, :] = zero16
        return carry
    lax.fori_loop(0, C, _zp, 0)
    for i in range(ROWS_PER_TILE // C):
        pltpu.sync_copy(posr, C_sp.at[pl.ds(tbase + C * i, C)])

    one0 = jnp.where(lax.iota(jnp.int32, 16) == 0, 1.0, 0.0).astype(jnp.float32)

    def _os(r, carry):
        ones_b[r, :] = one0
        return carry
    lax.fori_loop(0, C, _os, 0)

    pltpu.sync_copy(w1c_hbm, w1c_v)
    plsc.subcore_barrier()

    w1cs = [w1c_v[pl.ds(16 * j, 16)] for j in range(8)]
    lane = lax.iota(jnp.int32, 16)
    cidx0 = jnp.zeros((16,), jnp.int32)
    cidx1 = jnp.full((16,), 1, jnp.int32)
    cidx2 = jnp.full((16,), 2, jnp.int32)
    ebase = wid * EPW

    def _chunk(i, carry):
        base = ebase + i * C
        pltpu.sync_copy(row_hbm.at[pl.ds(base, C)], row_v)
        pltpu.sync_copy(col_hbm.at[pl.ds(base, C)], col_v)
        cpA = pltpu.async_copy(ap_hbm.at[row_v], bufA, semA)
        cpB = pltpu.async_copy(bt_hbm.at[col_v], bufB, semB)
        cpR = pltpu.async_copy(pos_hbm.at[row_v], posr, semC)
        cpC = pltpu.async_copy(pos_hbm.at[col_v], posc, semD)
        cpA.wait()
        cpB.wait()
        cpR.wait()
        cpC.wait()

        def _group(g, gcarry):
            eids = g * 16 + lane
            dx = plsc.load_gather(posr, [eids, cidx0]) - plsc.load_gather(posc, [eids, cidx0])
            dy = plsc.load_gather(posr, [eids, cidx1]) - plsc.load_gather(posc, [eids, cidx1])
            dz = plsc.load_gather(posr, [eids, cidx2]) - plsc.load_gather(posc, [eids, cidx2])
            d2 = dx * dx + dy * dy + dz * dz
            ii = plsc.bitcast(d2, jnp.int32)
            ii = 0x5F3759DF - lax.shift_right_logical(ii, 1)
            r = plsc.bitcast(ii, jnp.float32)
            r = r * (1.5 - 0.5 * d2 * r * r)
            r = r * (1.5 - 0.5 * d2 * r * r)
            r = r * (1.5 - 0.5 * d2 * r * r)
            dist_b[...] = d2 * r
            for e in range(16):
                de = plsc.load_gather(dist_b, [jnp.full((16,), e, jnp.int32)])
                erow = g * 16 + e
                for j in range(8):
                    a = bufA[erow, pl.ds(16 * j, 16)]
                    b = bufB[erow, pl.ds(16 * j, 16)]
                    z = a + b + de * w1cs[j]
                    sbuf[erow, pl.ds(16 * j, 16)] = z / (1.0 + jnp.exp(-z))
            return gcarry
        lax.fori_loop(0, 8, _group, 0)

        pltpu.sync_copy(sbuf, S_sp.at[row_v], add=True)
        pltpu.sync_copy(ones_b, C_sp.at[row_v], add=True)
        return carry
    lax.fori_loop(0, NCHUNK, _chunk, 0)

    plsc.subcore_barrier()
    pltpu.sync_copy(S_sp.at[pl.ds(tbase, ROWS_PER_TILE)],
                    s_out.at[core, pl.ds(tbase, ROWS_PER_TILE)])
    pltpu.sync_copy(C_sp.at[pl.ds(tbase, ROWS_PER_TILE)],
                    c_out.at[core, pl.ds(tbase, ROWS_PER_TILE)])


_sc_edge = functools.partial(
    pl.kernel,
    out_type=[
        jax.ShapeDtypeStruct((2, N_PAD, H), jnp.float32),
        jax.ShapeDtypeStruct((2, N_PAD, 16), jnp.float32),
    ],
    mesh=plsc.VectorSubcoreMesh(core_axis_name="c", subcore_axis_name="s"),
    scratch_types=[
        pltpu.VMEM((C,), jnp.int32),
        pltpu.VMEM((C,), jnp.int32),
        pltpu.VMEM((C, H), jnp.float32),
        pltpu.VMEM((C, H), jnp.float32),
        pltpu.VMEM((C, 16), jnp.float32),
        pltpu.VMEM((C, 16), jnp.float32),
        pltpu.VMEM((C, H), jnp.float32),
        pltpu.VMEM((C, 16), jnp.float32),
        pltpu.VMEM((H,), jnp.float32),
        pltpu.VMEM((16,), jnp.float32),
        pltpu.VMEM_SHARED((N_PAD, H), jnp.float32),
        pltpu.VMEM_SHARED((N_PAD, 16), jnp.float32),
        pltpu.SemaphoreType.DMA,
        pltpu.SemaphoreType.DMA,
        pltpu.SemaphoreType.DMA,
        pltpu.SemaphoreType.DMA,
    ],
)(_sc_edge_body)


def _post_body(s0_ref, s1_ref, c0_ref, c1_ref, w2_ref, b2_ref,
               nw1_ref, nb1_ref, nw2_ref, nb2_ref, out_ref):
    S = s0_ref[...] + s1_ref[...]
    cnt = jnp.sum(c0_ref[...] + c1_ref[...], axis=1, keepdims=True)
    num = jnp.dot(S, w2_ref[...], preferred_element_type=jnp.float32) + cnt * b2_ref[...]
    scalar = num / jnp.maximum(cnt, 1.0)
    t = jnp.dot(scalar, nw1_ref[...], preferred_element_type=jnp.float32) + nb1_ref[...]
    t = t / (1.0 + jnp.exp(-t))
    out_ref[...] = jnp.dot(t, nw2_ref[...], preferred_element_type=jnp.float32) + nb2_ref[...]


def _post(s0, s1, c0, c1, w2, b2, nw1, nb1, nw2, nb2):
    R = 1000
    full = lambda i: (0, 0)
    return pl.pallas_call(
        _post_body,
        grid=(N // R,),
        in_specs=[
            pl.BlockSpec((R, H), lambda i: (i, 0)),
            pl.BlockSpec((R, H), lambda i: (i, 0)),
            pl.BlockSpec((R, 16), lambda i: (i, 0)),
            pl.BlockSpec((R, 16), lambda i: (i, 0)),
            pl.BlockSpec((H, H), full),
            pl.BlockSpec((1, H), full),
            pl.BlockSpec((H, H), full),
            pl.BlockSpec((1, H), full),
            pl.BlockSpec((H, H), full),
            pl.BlockSpec((1, H), full),
        ],
        out_specs=pl.BlockSpec((R, H), lambda i: (i, 0)),
        out_shape=jax.ShapeDtypeStruct((N, H), jnp.float32),
    )(s0, s1, c0, c1, w2, b2, nw1, nb1, nw2, nb2)


def kernel(node_feat, node_pos, batch, edge_index,
           msg_W1, msg_b1, msg_W2, msg_b2,
           nf_W1, nf_b1, nf_W2, nf_b2):
    del batch
    row = edge_index[0].astype(jnp.int32)
    col = edge_index[1].astype(jnp.int32)
    row_p = jnp.pad(row.reshape(NW, E // NW), ((0, 0), (0, EPW - E // NW)),
                    constant_values=N).reshape(E_PAD)
    col_p = jnp.pad(col.reshape(NW, E // NW), ((0, 0), (0, EPW - E // NW)),
                    constant_values=N).reshape(E_PAD)

    nf_pad = jnp.pad(node_feat, ((0, N_PAD - N), (0, 0)))
    pos16 = jnp.pad(node_pos, ((0, N_PAD - N), (0, 13)))

    ap, bt = _pre(nf_pad, msg_W1[:H], msg_W1[H:2 * H], msg_b1.reshape(1, H))
    w1c = msg_W1[2 * H]

    s_part, c_part = _sc_edge(ap, bt, pos16, row_p, col_p, w1c)

    out = _post(s_part[0, :N], s_part[1, :N], c_part[0, :N], c_part[1, :N],
                msg_W2, msg_b2.reshape(1, H),
                nf_W1, nf_b1.reshape(1, H),
                nf_W2, nf_b2.reshape(1, H))
    return out


# speedup vs baseline: 3.3661x; 3.3661x over previous
import functools
import jax
import jax.numpy as jnp
from jax import lax
from jax.experimental import pallas as pl
from jax.experimental.pallas import tpu as pltpu
from jax.experimental.pallas import tpu_sc as plsc

N = 10000
E = 320000
H = 128
N_PAD = 10240
NW = 32
C = 128
NCHUNK = 79
EPW = C * NCHUNK
E_PAD = NW * EPW
ROWS_PER_TILE = N_PAD // 16


def _pre_body(nf_ref, w1a_ref, w1b_ref, b1_ref, ap_ref, bt_ref):
    nf = nf_ref[...]
    ap_ref[...] = jnp.dot(nf, w1a_ref[...], preferred_element_type=jnp.float32) + b1_ref[...]
    bt_ref[...] = jnp.dot(nf, w1b_ref[...], preferred_element_type=jnp.float32)


def _pre(nf_pad, w1a, w1b, b1):
    R = 512
    return pl.pallas_call(
        _pre_body,
        grid=(N_PAD // R,),
        in_specs=[
            pl.BlockSpec((R, H), lambda i: (i, 0)),
            pl.BlockSpec((H, H), lambda i: (0, 0)),
            pl.BlockSpec((H, H), lambda i: (0, 0)),
            pl.BlockSpec((1, H), lambda i: (0, 0)),
        ],
        out_specs=[
            pl.BlockSpec((R, H), lambda i: (i, 0)),
            pl.BlockSpec((R, H), lambda i: (i, 0)),
        ],
        out_shape=[
            jax.ShapeDtypeStruct((N_PAD, H), jnp.float32),
            jax.ShapeDtypeStruct((N_PAD, H), jnp.float32),
        ],
    )(nf_pad, w1a, w1b, b1)


def _sc_edge_body(ap_hbm, bt_hbm, px_hbm, py_hbm, pz_hbm, row_hbm, col_hbm,
                  w1c_hbm, s_out,
                  row_v, col_v, bufA, bufB, pxr, pyr, pzr, pxc, pyc, pzc,
                  w1c_v, S_sp,
                  semA, semB, sem0, sem1, sem2, sem3, sem4, sem5):
    core = lax.axis_index("c")
    sub = lax.axis_index("s")
    wid = sub * 2 + core
    tbase = sub * ROWS_PER_TILE

    zero16 = jnp.zeros((16,), jnp.float32)

    def _zs(r, carry):
        for j in range(8):
            bufA[r, pl.ds(16 * j, 16)] = zero16
        return carry
    lax.fori_loop(0, C, _zs, 0)
    for i in range(ROWS_PER_TILE // C):
        pltpu.sync_copy(bufA, S_sp.at[pl.ds(tbase + C * i, C)])

    pltpu.sync_copy(w1c_hbm, w1c_v)
    plsc.subcore_barrier()

    w1cs = [w1c_v[pl.ds(16 * j, 16)] for j in range(8)]
    ebase = wid * EPW

    def _chunk(i, carry):
        base = ebase + i * C
        pltpu.sync_copy(row_hbm.at[pl.ds(base, C)], row_v)
        pltpu.sync_copy(col_hbm.at[pl.ds(base, C)], col_v)
        cps = [
            pltpu.async_copy(ap_hbm.at[row_v], bufA, semA),
            pltpu.async_copy(bt_hbm.at[col_v], bufB, semB),
            pltpu.async_copy(px_hbm.at[row_v], pxr, sem0),
            pltpu.async_copy(py_hbm.at[row_v], pyr, sem1),
            pltpu.async_copy(pz_hbm.at[row_v], pzr, sem2),
            pltpu.async_copy(px_hbm.at[col_v], pxc, sem3),
            pltpu.async_copy(py_hbm.at[col_v], pyc, sem4),
            pltpu.async_copy(pz_hbm.at[col_v], pzc, sem5),
        ]
        for cp in cps:
            cp.wait()

        def _group(g, gcarry):
            gb = g * 16
            dx = pxr[pl.ds(gb, 16)] - pxc[pl.ds(gb, 16)]
            dy = pyr[pl.ds(gb, 16)] - pyc[pl.ds(gb, 16)]
            dz = pzr[pl.ds(gb, 16)] - pzc[pl.ds(gb, 16)]
            d2 = dx * dx + dy * dy + dz * dz
            r = jnp.full((16,), 316.22776, jnp.float32)
            for t, v in ((1e-5, 100.0), (1e-4, 31.622776), (1e-3, 10.0),
                         (1e-2, 3.1622776), (1e-1, 1.0), (1e0, 0.31622776),
                         (1e1, 0.1), (1e2, 0.031622776), (1e3, 0.01)):
                r = jnp.where(d2 >= t, v, r)
            for _ in range(7):
                r = r * (1.5 - 0.5 * d2 * r * r)
            dist = d2 * r
            for e in range(16):
                de = jnp.full((16,), dist[e], jnp.float32)
                erow = gb + e
                for j in range(8):
                    a = bufA[erow, pl.ds(16 * j, 16)]
                    b = bufB[erow, pl.ds(16 * j, 16)]
                    z = a + b + de * w1cs[j]
                    bufA[erow, pl.ds(16 * j, 16)] = z / (1.0 + jnp.exp(-z))
            return gcarry
        lax.fori_loop(0, C // 16, _group, 0)

        pltpu.sync_copy(bufA, S_sp.at[row_v], add=True)
        return carry
    lax.fori_loop(0, NCHUNK, _chunk, 0)

    plsc.subcore_barrier()
    pltpu.sync_copy(S_sp.at[pl.ds(tbase, ROWS_PER_TILE)],
                    s_out.at[core, pl.ds(tbase, ROWS_PER_TILE)])


_sc_edge = functools.partial(
    pl.kernel,
    out_type=jax.ShapeDtypeStruct((2, N_PAD, H), jnp.float32),
    mesh=plsc.VectorSubcoreMesh(core_axis_name="c", subcore_axis_name="s"),
    scratch_types=[
        pltpu.VMEM((C,), jnp.int32),
        pltpu.VMEM((C,), jnp.int32),
        pltpu.VMEM((C, H), jnp.float32),
        pltpu.VMEM((C, H), jnp.float32),
        pltpu.VMEM((C,), jnp.float32),
        pltpu.VMEM((C,), jnp.float32),
        pltpu.VMEM((C,), jnp.float32),
        pltpu.VMEM((C,), jnp.float32),
        pltpu.VMEM((C,), jnp.float32),
        pltpu.VMEM((C,), jnp.float32),
        pltpu.VMEM((H,), jnp.float32),
        pltpu.VMEM_SHARED((N_PAD, H), jnp.float32),
    ] + [pltpu.SemaphoreType.DMA] * 8,
)(_sc_edge_body)


def _sc_cnt_body(row_hbm, c_out, row_v, ones_b, C_sp):
    core = lax.axis_index("c")
    sub = lax.axis_index("s")
    wid = sub * 2 + core
    tbase = sub * ROWS_PER_TILE

    zero16 = jnp.zeros((16,), jnp.float32)
    one0 = jnp.where(lax.iota(jnp.int32, 16) == 0, 1.0, 0.0).astype(jnp.float32)

    def _zp(r, carry):
        for j in range(H // 16):
            ones_b[r, pl.ds(16 * j, 16)] = zero16
        return carry
    lax.fori_loop(0, C, _zp, 0)
    for i in range(ROWS_PER_TILE // C):
        pltpu.sync_copy(ones_b, C_sp.at[pl.ds(tbase + C * i, C)])

    def _os(r, carry):
        ones_b[r, pl.ds(0, 16)] = one0
        return carry
    lax.fori_loop(0, C, _os, 0)
    plsc.subcore_barrier()

    ebase = wid * EPW

    def _chunk(i, carry):
        pltpu.sync_copy(row_hbm.at[pl.ds(ebase + i * C, C)], row_v)
        pltpu.sync_copy(ones_b, C_sp.at[row_v], add=True)
        return carry
    lax.fori_loop(0, NCHUNK, _chunk, 0)

    plsc.subcore_barrier()
    pltpu.sync_copy(C_sp.at[pl.ds(tbase, ROWS_PER_TILE)],
                    c_out.at[core, pl.ds(tbase, ROWS_PER_TILE)])


_sc_cnt = functools.partial(
    pl.kernel,
    out_type=jax.ShapeDtypeStruct((2, N_PAD, H), jnp.float32),
    mesh=plsc.VectorSubcoreMesh(core_axis_name="c", subcore_axis_name="s"),
    scratch_types=[
        pltpu.VMEM((C,), jnp.int32),
        pltpu.VMEM((C, H), jnp.float32),
        pltpu.VMEM_SHARED((N_PAD, H), jnp.float32),
    ],
)(_sc_cnt_body)


def _post_body(s0_ref, s1_ref, c0_ref, c1_ref, w2_ref, b2_ref,
               nw1_ref, nb1_ref, nw2_ref, nb2_ref, out_ref):
    S = s0_ref[...] + s1_ref[...]
    cnt = jnp.sum(c0_ref[...] + c1_ref[...], axis=1, keepdims=True)
    num = jnp.dot(S, w2_ref[...], preferred_element_type=jnp.float32) + cnt * b2_ref[...]
    scalar = num / jnp.maximum(cnt, 1.0)
    t = jnp.dot(scalar, nw1_ref[...], preferred_element_type=jnp.float32) + nb1_ref[...]
    t = t / (1.0 + jnp.exp(-t))
    out_ref[...] = jnp.dot(t, nw2_ref[...], preferred_element_type=jnp.float32) + nb2_ref[...]


def _post(s0, s1, c0, c1, w2, b2, nw1, nb1, nw2, nb2):
    R = 1000
    full = lambda i: (0, 0)
    return pl.pallas_call(
        _post_body,
        grid=(N // R,),
        in_specs=[
            pl.BlockSpec((R, H), lambda i: (i, 0)),
            pl.BlockSpec((R, H), lambda i: (i, 0)),
            pl.BlockSpec((R, H), lambda i: (i, 0)),
            pl.BlockSpec((R, H), lambda i: (i, 0)),
            pl.BlockSpec((H, H), full),
            pl.BlockSpec((1, H), full),
            pl.BlockSpec((H, H), full),
            pl.BlockSpec((1, H), full),
            pl.BlockSpec((H, H), full),
            pl.BlockSpec((1, H), full),
        ],
        out_specs=pl.BlockSpec((R, H), lambda i: (i, 0)),
        out_shape=jax.ShapeDtypeStruct((N, H), jnp.float32),
    )(s0, s1, c0, c1, w2, b2, nw1, nb1, nw2, nb2)


def kernel(node_feat, node_pos, batch, edge_index,
           msg_W1, msg_b1, msg_W2, msg_b2,
           nf_W1, nf_b1, nf_W2, nf_b2):
    del batch
    row = edge_index[0].astype(jnp.int32)
    col = edge_index[1].astype(jnp.int32)
    row_p = jnp.pad(row.reshape(NW, E // NW), ((0, 0), (0, EPW - E // NW)),
                    constant_values=N).reshape(E_PAD)
    col_p = jnp.pad(col.reshape(NW, E // NW), ((0, 0), (0, EPW - E // NW)),
                    constant_values=N).reshape(E_PAD)

    nf_pad = jnp.pad(node_feat, ((0, N_PAD - N), (0, 0)))
    pos_pad = jnp.pad(node_pos, ((0, N_PAD - N), (0, 0)))
    px = pos_pad[:, 0] + 0.0
    py = pos_pad[:, 1] + 0.0
    pz = pos_pad[:, 2] + 0.0

    ap, bt = _pre(nf_pad, msg_W1[:H], msg_W1[H:2 * H], msg_b1.reshape(1, H))
    w1c = msg_W1[2 * H]

    s_part = _sc_edge(ap, bt, px, py, pz, row_p, col_p, w1c)
    c_part = _sc_cnt(row_p)

    out = _post(s_part[0, :N], s_part[1, :N], c_part[0, :N], c_part[1, :N],
                msg_W2, msg_b2.reshape(1, H),
                nf_W1, nf_b1.reshape(1, H),
                nf_W2, nf_b2.reshape(1, H))
    return out


# speedup vs baseline: 3.5879x; 1.0659x over previous
import functools
import jax
import jax.numpy as jnp
from jax import lax
from jax.experimental import pallas as pl
from jax.experimental.pallas import tpu as pltpu
from jax.experimental.pallas import tpu_sc as plsc

N = 10000
E = 320000
H = 128
N_PAD = 10240
NW = 32
C = 128
NCHUNK = 79
CE = 64
NCHUNK_E = 158
EPW = C * NCHUNK
E_PAD = NW * EPW
ROWS_PER_TILE = N_PAD // 16


def _pre_body(nf_ref, w1a_ref, w1b_ref, b1_ref, ap_ref, bt_ref):
    nf = nf_ref[...]
    ap_ref[...] = jnp.dot(nf, w1a_ref[...], preferred_element_type=jnp.float32) + b1_ref[...]
    bt_ref[...] = jnp.dot(nf, w1b_ref[...], preferred_element_type=jnp.float32)


def _pre(nf_pad, w1a, w1b, b1):
    R = 512
    return pl.pallas_call(
        _pre_body,
        grid=(N_PAD // R,),
        in_specs=[
            pl.BlockSpec((R, H), lambda i: (i, 0)),
            pl.BlockSpec((H, H), lambda i: (0, 0)),
            pl.BlockSpec((H, H), lambda i: (0, 0)),
            pl.BlockSpec((1, H), lambda i: (0, 0)),
        ],
        out_specs=[
            pl.BlockSpec((R, H), lambda i: (i, 0)),
            pl.BlockSpec((R, H), lambda i: (i, 0)),
        ],
        out_shape=[
            jax.ShapeDtypeStruct((N_PAD, H), jnp.float32),
            jax.ShapeDtypeStruct((N_PAD, H), jnp.float32),
        ],
    )(nf_pad, w1a, w1b, b1)


def _sc_edge_body(ap_hbm, bt_hbm, px_hbm, py_hbm, pz_hbm, rc_hbm, w1c_hbm,
                  s_out,
                  rc0, rc1, bufA0, bufA1, bufB0, bufB1,
                  pxr0, pxr1, pyr0, pyr1, pzr0, pzr1,
                  pxc0, pxc1, pyc0, pyc1, pzc0, pzc1,
                  w1c_v, S_sp, sem0, sem1):
    core = lax.axis_index("c")
    sub = lax.axis_index("s")
    wid = sub * 2 + core
    tbase = sub * ROWS_PER_TILE

    rc = (rc0, rc1)
    bufA = (bufA0, bufA1)
    bufB = (bufB0, bufB1)
    pxr = (pxr0, pxr1)
    pyr = (pyr0, pyr1)
    pzr = (pzr0, pzr1)
    pxc = (pxc0, pxc1)
    pyc = (pyc0, pyc1)
    pzc = (pzc0, pzc1)
    sem = (sem0, sem1)

    zero16 = jnp.zeros((16,), jnp.float32)

    def _zs(r, carry):
        for j in range(8):
            bufA0[r, pl.ds(16 * j, 16)] = zero16
        return carry
    lax.fori_loop(0, CE, _zs, 0)
    for i in range(ROWS_PER_TILE // CE):
        pltpu.sync_copy(bufA0, S_sp.at[pl.ds(tbase + CE * i, CE)])

    pltpu.sync_copy(w1c_hbm, w1c_v)

    rcbase = wid * NCHUNK_E * 2

    def _issue(i, b):
        pltpu.sync_copy(rc_hbm.at[pl.ds(rcbase + i * 2, 2)], rc[b])
        rv = rc[b].at[0]
        cv = rc[b].at[1]
        pltpu.async_copy(ap_hbm.at[rv], bufA[b], sem[b])
        pltpu.async_copy(bt_hbm.at[cv], bufB[b], sem[b])
        pltpu.async_copy(px_hbm.at[rv], pxr[b], sem[b])
        pltpu.async_copy(py_hbm.at[rv], pyr[b], sem[b])
        pltpu.async_copy(pz_hbm.at[rv], pzr[b], sem[b])
        pltpu.async_copy(px_hbm.at[cv], pxc[b], sem[b])
        pltpu.async_copy(py_hbm.at[cv], pyc[b], sem[b])
        pltpu.async_copy(pz_hbm.at[cv], pzc[b], sem[b])

    def _drain(b):
        rv = rc[b].at[0]
        cv = rc[b].at[1]
        pltpu.make_async_copy(ap_hbm.at[rv], bufA[b], sem[b]).wait()
        pltpu.make_async_copy(bt_hbm.at[cv], bufB[b], sem[b]).wait()
        pltpu.make_async_copy(px_hbm.at[rv], pxr[b], sem[b]).wait()
        pltpu.make_async_copy(py_hbm.at[rv], pyr[b], sem[b]).wait()
        pltpu.make_async_copy(pz_hbm.at[rv], pzr[b], sem[b]).wait()
        pltpu.make_async_copy(px_hbm.at[cv], pxc[b], sem[b]).wait()
        pltpu.make_async_copy(py_hbm.at[cv], pyc[b], sem[b]).wait()
        pltpu.make_async_copy(pz_hbm.at[cv], pzc[b], sem[b]).wait()

    _issue(0, 0)
    _issue(1, 1)
    plsc.subcore_barrier()

    w1cs = [w1c_v[pl.ds(16 * j, 16)] for j in range(8)]

    def _compute(b):
        def _group(g, gcarry):
            gb = g * 16
            dx = pxr[b][pl.ds(gb, 16)] - pxc[b][pl.ds(gb, 16)]
            dy = pyr[b][pl.ds(gb, 16)] - pyc[b][pl.ds(gb, 16)]
            dz = pzr[b][pl.ds(gb, 16)] - pzc[b][pl.ds(gb, 16)]
            d2 = dx * dx + dy * dy + dz * dz
            r = jnp.full((16,), 316.22776, jnp.float32)
            for t, v in ((1e-5, 100.0), (1e-4, 31.622776), (1e-3, 10.0),
                         (1e-2, 3.1622776), (1e-1, 1.0), (1e0, 0.31622776),
                         (1e1, 0.1), (1e2, 0.031622776), (1e3, 0.01)):
                r = jnp.where(d2 >= t, v, r)
            for _ in range(7):
                r = r * (1.5 - 0.5 * d2 * r * r)
            dist = d2 * r
            for e in range(16):
                de = jnp.full((16,), dist[e], jnp.float32)
                erow = gb + e
                for j in range(8):
                    a = bufA[b][erow, pl.ds(16 * j, 16)]
                    bb = bufB[b][erow, pl.ds(16 * j, 16)]
                    z = a + bb + de * w1cs[j]
                    bufA[b][erow, pl.ds(16 * j, 16)] = z / (1.0 + jnp.exp(-z))
            return gcarry
        lax.fori_loop(0, CE // 16, _group, 0)

    def _outer(i2, carry):
        for b in range(2):
            i = i2 * 2 + b
            _drain(b)
            _compute(b)
            pltpu.sync_copy(bufA[b], S_sp.at[rc[b].at[0]], add=True)

            @pl.when(i + 2 < NCHUNK_E)
            def _():
                _issue(i + 2, b)
        return carry
    lax.fori_loop(0, NCHUNK_E // 2, _outer, 0)

    plsc.subcore_barrier()
    pltpu.sync_copy(S_sp.at[pl.ds(tbase, ROWS_PER_TILE)],
                    s_out.at[core, pl.ds(tbase, ROWS_PER_TILE)])


_sc_edge = functools.partial(
    pl.kernel,
    out_type=jax.ShapeDtypeStruct((2, N_PAD, H), jnp.float32),
    mesh=plsc.VectorSubcoreMesh(core_axis_name="c", subcore_axis_name="s"),
    scratch_types=(
        [pltpu.VMEM((2, CE), jnp.int32)] * 2 +
        [pltpu.VMEM((CE, H), jnp.float32)] * 4 +
        [pltpu.VMEM((CE,), jnp.float32)] * 12 +
        [pltpu.VMEM((H,), jnp.float32)] +
        [pltpu.VMEM_SHARED((N_PAD, H), jnp.float32)] +
        [pltpu.SemaphoreType.DMA] * 2
    ),
)(_sc_edge_body)


def _sc_cnt_body(row_hbm, c_out, row_v, ones_b, C_sp):
    core = lax.axis_index("c")
    sub = lax.axis_index("s")
    wid = sub * 2 + core
    tbase = sub * ROWS_PER_TILE

    zero16 = jnp.zeros((16,), jnp.float32)
    one0 = jnp.where(lax.iota(jnp.int32, 16) == 0, 1.0, 0.0).astype(jnp.float32)

    def _zp(r, carry):
        for j in range(H // 16):
            ones_b[r, pl.ds(16 * j, 16)] = zero16
        return carry
    lax.fori_loop(0, C, _zp, 0)
    for i in range(ROWS_PER_TILE // C):
        pltpu.sync_copy(ones_b, C_sp.at[pl.ds(tbase + C * i, C)])

    def _os(r, carry):
        ones_b[r, pl.ds(0, 16)] = one0
        return carry
    lax.fori_loop(0, C, _os, 0)
    plsc.subcore_barrier()

    ebase = wid * EPW

    def _chunk(i, carry):
        pltpu.sync_copy(row_hbm.at[pl.ds(ebase + i * C, C)], row_v)
        pltpu.sync_copy(ones_b, C_sp.at[row_v], add=True)
        return carry
    lax.fori_loop(0, NCHUNK, _chunk, 0)

    plsc.subcore_barrier()
    pltpu.sync_copy(C_sp.at[pl.ds(tbase, ROWS_PER_TILE)],
                    c_out.at[core, pl.ds(tbase, ROWS_PER_TILE)])


_sc_cnt = functools.partial(
    pl.kernel,
    out_type=jax.ShapeDtypeStruct((2, N_PAD, H), jnp.float32),
    mesh=plsc.VectorSubcoreMesh(core_axis_name="c", subcore_axis_name="s"),
    scratch_types=[
        pltpu.VMEM((C,), jnp.int32),
        pltpu.VMEM((C, H), jnp.float32),
        pltpu.VMEM_SHARED((N_PAD, H), jnp.float32),
    ],
)(_sc_cnt_body)


def _post_body(s0_ref, s1_ref, c0_ref, c1_ref, w2_ref, b2_ref,
               nw1_ref, nb1_ref, nw2_ref, nb2_ref, out_ref):
    S = s0_ref[...] + s1_ref[...]
    cnt = jnp.sum(c0_ref[...] + c1_ref[...], axis=1, keepdims=True)
    num = jnp.dot(S, w2_ref[...], preferred_element_type=jnp.float32) + cnt * b2_ref[...]
    scalar = num / jnp.maximum(cnt, 1.0)
    t = jnp.dot(scalar, nw1_ref[...], preferred_element_type=jnp.float32) + nb1_ref[...]
    t = t / (1.0 + jnp.exp(-t))
    out_ref[...] = jnp.dot(t, nw2_ref[...], preferred_element_type=jnp.float32) + nb2_ref[...]


def _post(s0, s1, c0, c1, w2, b2, nw1, nb1, nw2, nb2):
    R = 1000
    full = lambda i: (0, 0)
    return pl.pallas_call(
        _post_body,
        grid=(N // R,),
        in_specs=[
            pl.BlockSpec((R, H), lambda i: (i, 0)),
            pl.BlockSpec((R, H), lambda i: (i, 0)),
            pl.BlockSpec((R, H), lambda i: (i, 0)),
            pl.BlockSpec((R, H), lambda i: (i, 0)),
            pl.BlockSpec((H, H), full),
            pl.BlockSpec((1, H), full),
            pl.BlockSpec((H, H), full),
            pl.BlockSpec((1, H), full),
            pl.BlockSpec((H, H), full),
            pl.BlockSpec((1, H), full),
        ],
        out_specs=pl.BlockSpec((R, H), lambda i: (i, 0)),
        out_shape=jax.ShapeDtypeStruct((N, H), jnp.float32),
    )(s0, s1, c0, c1, w2, b2, nw1, nb1, nw2, nb2)


def kernel(node_feat, node_pos, batch, edge_index,
           msg_W1, msg_b1, msg_W2, msg_b2,
           nf_W1, nf_b1, nf_W2, nf_b2):
    del batch
    row = edge_index[0].astype(jnp.int32)
    col = edge_index[1].astype(jnp.int32)
    row_p = jnp.pad(row.reshape(NW, E // NW), ((0, 0), (0, EPW - E // NW)),
                    constant_values=N).reshape(E_PAD)
    col_p = jnp.pad(col.reshape(NW, E // NW), ((0, 0), (0, EPW - E // NW)),
                    constant_values=N).reshape(E_PAD)
    rc = jnp.stack([row_p.reshape(NW * NCHUNK_E, CE),
                    col_p.reshape(NW * NCHUNK_E, CE)], axis=1).reshape(-1, CE)

    nf_pad = jnp.pad(node_feat, ((0, N_PAD - N), (0, 0)))
    pos_pad = jnp.pad(node_pos, ((0, N_PAD - N), (0, 0)))
    px = pos_pad[:, 0] + 0.0
    py = pos_pad[:, 1] + 0.0
    pz = pos_pad[:, 2] + 0.0

    ap, bt = _pre(nf_pad, msg_W1[:H], msg_W1[H:2 * H], msg_b1.reshape(1, H))
    w1c = msg_W1[2 * H]

    s_part = _sc_edge(ap, bt, px, py, pz, rc, w1c)
    c_part = _sc_cnt(row_p)

    out = _post(s_part[0, :N], s_part[1, :N], c_part[0, :N], c_part[1, :N],
                msg_W2, msg_b2.reshape(1, H),
                nf_W1, nf_b1.reshape(1, H),
                nf_W2, nf_b2.reshape(1, H))
    return out
